# baseline (speedup 1.0000x reference)
import jax
import jax.numpy as jnp
from jax import lax
from jax.experimental import pallas as pl
from jax.experimental.pallas import tpu as pltpu
from jax.experimental.pallas import tpu_sc as plsc

N_TOKENS = 32768
INPUT_DIM = 768
N_EXPERTS = 64
BLOCK_T = 4096

NC = 2
NS = 16
L = 16
NW = NC * NS
TPW = N_TOKENS // NW
NG = TPW // L


def _logits_body(x_ref, w_ref, b_ref, out_ref):
    out_ref[...] = lax.dot_general(
        w_ref[...], x_ref[...], (((1,), (1,)), ((), ())),
        preferred_element_type=jnp.float32) + b_ref[...]


def _logits_t(x, W, b):
    grid = (N_TOKENS // BLOCK_T,)
    return pl.pallas_call(
        _logits_body,
        grid=grid,
        in_specs=[
            pl.BlockSpec((BLOCK_T, INPUT_DIM), lambda i: (i, 0)),
            pl.BlockSpec((N_EXPERTS, INPUT_DIM), lambda i: (0, 0)),
            pl.BlockSpec((N_EXPERTS, 1), lambda i: (0, 0)),
        ],
        out_specs=pl.BlockSpec((N_EXPERTS, BLOCK_T), lambda i: (0, i)),
        out_shape=jax.ShapeDtypeStruct((N_EXPERTS, N_TOKENS), jnp.float32),
    )(x, W, b.reshape(N_EXPERTS, 1))


def _route_body(lg_hbm, gate_hbm, idx_hbm, lt, gv, iv):
    wid = lax.axis_index("s") * NC + lax.axis_index("c")
    base = wid * TPW
    pltpu.sync_copy(lg_hbm.at[:, pl.ds(base, TPW)], lt)
    lane = lax.iota(jnp.int32, L)

    ilp = 8

    def super_group(sg, carry):
        neg = jnp.full((L,), -jnp.inf, jnp.float32)
        zero = jnp.zeros((L,), jnp.int32)
        v1 = [neg] * ilp
        v2 = [neg] * ilp
        i1 = [zero] * ilp
        i2 = [zero] * ilp
        for e in range(N_EXPERTS):
            ei = jnp.full((L,), e, jnp.int32)
            for k in range(ilp):
                v = lt[e, pl.ds(sg * (ilp * L) + k * L, L)]
                gt1 = v > v1[k]
                gt2 = v > v2[k]
                lo = jnp.minimum(v1[k], v)
                i2[k] = jnp.where(gt1, i1[k],
                                  jnp.where(gt2, ei, i2[k]))
                i1[k] = jnp.where(gt1, ei, i1[k])
                v2[k] = jnp.maximum(v2[k], lo)
                v1[k] = jnp.maximum(v1[k], v)
        for k in range(ilp):
            p1 = 1.0 / (1.0 + jnp.exp(v2[k] - v1[k]))
            off = sg * 256 + k * L
            gv[pl.ds(off, L)] = p1
            gv[pl.ds(off + 128, L)] = 1.0 - p1
            iv[pl.ds(off, L)] = i1[k]
            iv[pl.ds(off + 128, L)] = i2[k]
        return carry

    lax.fori_loop(0, NG // ilp, super_group, 0)
    pltpu.sync_copy(gv, gate_hbm.at[pl.ds(2 * base, 2 * TPW)])
    pltpu.sync_copy(iv, idx_hbm.at[pl.ds(2 * base, 2 * TPW)])


EPI_T = 8192


def _epilogue_body(g_ref, i_ref, gp_ref, idx_ref):
    rows = EPI_T // 128
    g3 = g_ref[...].reshape(rows, 2, 128)
    i3 = i_ref[...].reshape(rows, 2, 128)
    p1 = g3[:, 0, :].reshape(EPI_T, 1)
    p2 = g3[:, 1, :].reshape(EPI_T, 1)
    gp_ref[...] = jnp.concatenate([p1, p2], axis=1)
    j1 = i3[:, 0, :].reshape(EPI_T, 1)
    j2 = i3[:, 1, :].reshape(EPI_T, 1)
    idx_ref[...] = jnp.concatenate([j1, j2], axis=1)


def _epilogue(gate128, idx128):
    grid = (N_TOKENS // EPI_T,)
    rows = 2 * EPI_T // 128
    return pl.pallas_call(
        _epilogue_body,
        grid=grid,
        in_specs=[
            pl.BlockSpec((rows, 128), lambda i: (i, 0)),
            pl.BlockSpec((rows, 128), lambda i: (i, 0)),
        ],
        out_specs=[
            pl.BlockSpec((EPI_T, 2), lambda i: (i, 0)),
            pl.BlockSpec((EPI_T, 2), lambda i: (i, 0)),
        ],
        out_shape=[
            jax.ShapeDtypeStruct((N_TOKENS, 2), jnp.float32),
            jax.ShapeDtypeStruct((N_TOKENS, 2), jnp.int32),
        ],
    )(gate128, idx128)


def kernel(x, W, b):
    lg = _logits_t(x, W, b)
    mesh = plsc.VectorSubcoreMesh(
        core_axis_name="c", subcore_axis_name="s",
        num_cores=NC, num_subcores=NS)
    route = pl.kernel(
        _route_body,
        out_type=[
            jax.ShapeDtypeStruct((2 * N_TOKENS,), jnp.float32),
            jax.ShapeDtypeStruct((2 * N_TOKENS,), jnp.int32),
        ],
        mesh=mesh,
        scratch_types=[
            pltpu.VMEM((N_EXPERTS, TPW), jnp.float32),
            pltpu.VMEM((2 * TPW,), jnp.float32),
            pltpu.VMEM((2 * TPW,), jnp.int32),
        ],
        compiler_params=pltpu.CompilerParams(needs_layout_passes=False),
    )
    gate128, idx128 = route(lg)
    gp = gate128.reshape(256, 2, 128).transpose(0, 2, 1).reshape(N_TOKENS, 2)
    ii = idx128.reshape(256, 2, 128).transpose(0, 2, 1).reshape(N_TOKENS, 2)
    return (gp, ii)
